# baseline (speedup 1.0000x reference)
import functools

import jax
import jax.numpy as jnp
from jax import lax
from jax.experimental import pallas as pl
from jax.experimental.pallas import tpu as pltpu
from jax.experimental.pallas import tpu_sc as plsc

_VOCAB = 100000
_EMBED = 128
_OUT = 64
_BATCH = 4096
_SEQ = 200
_LANES = 16

_CHUNK = 104
_NCHUNK = 2
_SEQ_PAD = _CHUNK * _NCHUNK

_PROJ_BLK = 2000

_info = plsc.get_sparse_core_info()
_NC, _NS = _info.num_cores, _info.num_subcores
_NW = _NC * _NS
_BPW = _BATCH // _NW


def _proj_body(emb_ref, w_ref, out_ref):
    blk = lax.dot_general(
        emb_ref[...], w_ref[...],
        dimension_numbers=(((1,), (1,)), ((), ())),
        preferred_element_type=jnp.float32,
    ) * (1.0 / _SEQ)
    row = (lax.broadcasted_iota(jnp.int32, blk.shape, 0)
           + pl.program_id(0) * _PROJ_BLK)
    out_ref[...] = jnp.where(row == 0, 0.0, blk)


def _project(emb, w):
    return pl.pallas_call(
        _proj_body,
        grid=(_VOCAB // _PROJ_BLK,),
        in_specs=[
            pl.BlockSpec((_PROJ_BLK, _EMBED), lambda i: (i, 0)),
            pl.BlockSpec((_OUT, _EMBED), lambda i: (0, 0)),
        ],
        out_specs=pl.BlockSpec((_PROJ_BLK, _OUT), lambda i: (i, 0)),
        out_shape=jax.ShapeDtypeStruct((_VOCAB, _OUT), jnp.float32),
    )(emb, w)


_NBUF = 8
_RPB = _NBUF // _NCHUNK
_NBLK = _BPW // _RPB


@functools.partial(
    pl.kernel,
    out_type=jax.ShapeDtypeStruct((_BATCH, _OUT), jnp.float32),
    mesh=plsc.VectorSubcoreMesh(core_axis_name="c", subcore_axis_name="s"),
    compiler_params=pltpu.CompilerParams(use_tc_tiling_on_sc=False),
    scratch_types=[
        pltpu.VMEM((_BPW * _SEQ_PAD,), jnp.int32),
        pltpu.VMEM((_NBUF, _CHUNK, _OUT), jnp.float32),
        pltpu.VMEM((_BPW, _OUT), jnp.float32),
        pltpu.VMEM((_OUT,), jnp.float32),
        [pltpu.SemaphoreType.DMA] * _NBUF,
    ],
)
def _pool(idx_hbm, p_hbm, b_hbm, out_hbm, idx_v, rows_v, out_v, bias_v, sems):
    wid = lax.axis_index("s") * _NC + lax.axis_index("c")
    base = wid * _BPW
    pltpu.sync_copy(idx_hbm.at[pl.ds(base * _SEQ_PAD, _BPW * _SEQ_PAD)], idx_v)
    pltpu.sync_copy(b_hbm, bias_v)

    def _start(i, j, slot):
        off = pl.multiple_of(i * _SEQ_PAD + j * _CHUNK, 8)
        pltpu.make_async_copy(
            p_hbm.at[idx_v.at[pl.ds(off, _CHUNK)]],
            rows_v.at[slot],
            sems[slot],
        ).start()

    def _wait(slot):
        pltpu.make_async_copy(
            p_hbm.at[idx_v.at[pl.ds(0, _CHUNK)]],
            rows_v.at[slot],
            sems[slot],
        ).wait()

    for b in range(_NBUF):
        _start(b // _NCHUNK, b % _NCHUNK, b)

    def blk_body(kk, carry):
        for r in range(_RPB):
            i = kk * _RPB + r
            s0, s1 = _NCHUNK * r, _NCHUNK * r + 1
            _wait(s0)
            _wait(s1)

            def s_body(s, accs):
                return tuple(
                    accs[k]
                    + rows_v[s0, s, pl.ds(_LANES * k, _LANES)]
                    + rows_v[s1, s, pl.ds(_LANES * k, _LANES)]
                    for k in range(_OUT // _LANES))

            accs = lax.fori_loop(
                0, _CHUNK, s_body,
                tuple(bias_v[pl.ds(_LANES * k, _LANES)]
                      for k in range(_OUT // _LANES)),
                unroll=4,
            )
            for k in range(_OUT // _LANES):
                out_v[i, pl.ds(_LANES * k, _LANES)] = accs[k]

            @pl.when(kk < _NBLK - 1)
            def _():
                _start(i + _RPB, 0, s0)
                _start(i + _RPB, 1, s1)

        return carry

    lax.fori_loop(0, _NBLK, blk_body, 0)
    pltpu.sync_copy(out_v, out_hbm.at[pl.ds(base, _BPW)])


def kernel(x, emb, W, b):
    idx = jnp.pad(x.astype(jnp.int32), ((0, 0), (0, _SEQ_PAD - _SEQ)))
    p = _project(emb, W)
    return _pool(idx.reshape(-1), p, b)
